# baseline (speedup 1.0000x reference)
import jax
import jax.numpy as jnp
from jax import lax
from jax.experimental import pallas as pl
from jax.experimental.pallas import tpu as pltpu
from jax.experimental.pallas import tpu_sc as plsc

NC = 2
NS = 16
C = 128


def _sc_hop(table, packed, zacc, *, np_, half, epad):
    d = table.shape[1]
    nploc = half + C
    per_tile = epad // NS
    nchunks = per_tile // C
    zr = nploc // NS
    dr = half // NS
    NB = 4

    mesh = plsc.VectorSubcoreMesh(core_axis_name="c", subcore_axis_name="s")

    def body(table_hbm, packed_hbm, zacc_hbm, out_hbm, acc_sh, *rest):
        idx = rest[0:NB]
        sadj = rest[NB:2 * NB]
        rows = rest[2 * NB:3 * NB]
        isem = rest[3 * NB:4 * NB]
        gsem = rest[4 * NB:5 * NB]
        ssem = rest[5 * NB:6 * NB]
        c = lax.axis_index("c")
        s = lax.axis_index("s")

        pltpu.sync_copy(zacc_hbm.at[pl.ds(s * zr, zr)],
                        acc_sh.at[pl.ds(s * zr, zr)])
        plsc.subcore_barrier()

        base = s * per_tile
        lo = c * half

        def idx_src(k):
            return packed_hbm.at[pl.ds((base + k * C) * 2, 2 * C)]

        def issue_idx(k, b):
            pltpu.async_copy(idx_src(k), idx[b], isem[b])

        def prep_and_gather(k, b):
            pltpu.make_async_copy(idx_src(k), idx[b], isem[b]).wait()
            for i in range(C // 16):
                v = idx[b][pl.ds(C + i * 16, 16)] - lo
                ok = (v >= 0) & (v < half)
                sadj[b][pl.ds(i * 16, 16)] = jnp.where(ok, v, half)
            pltpu.async_copy(table_hbm.at[idx[b].at[pl.ds(0, C)]],
                             rows[b], gsem[b])

        def wait_scatter(b):
            pltpu.make_async_copy(rows[b], acc_sh.at[sadj[b]],
                                  ssem[b]).wait()

        def issue_scatter(b):
            pltpu.make_async_copy(rows[b], acc_sh.at[sadj[b]],
                                  gsem[b]).wait()
            pltpu.async_copy(rows[b], acc_sh.at[sadj[b]], ssem[b],
                             add=True)

        issue_idx(0, 0)
        issue_idx(1, 1)
        prep_and_gather(0, 0)

        P = nchunks // NB

        def step(p, carry):
            for b in range(NB):
                k = 4 * p + b
                bn_ = (b + 1) % NB
                if b == NB - 1:
                    @pl.when(p < P - 1)
                    def _():
                        wait_scatter(bn_)
                        prep_and_gather(k + 1, bn_)
                elif b == NB - 2:
                    @pl.when(p > 0)
                    def _():
                        wait_scatter(bn_)
                    prep_and_gather(k + 1, bn_)
                else:
                    @pl.when(p > 0)
                    def _():
                        wait_scatter(bn_)
                    prep_and_gather(k + 1, bn_)
                b2 = (b + 2) % NB
                if b >= NB - 2:
                    @pl.when(p < P - 1)
                    def _():
                        issue_idx(k + 2, b2)
                else:
                    issue_idx(k + 2, b2)
                issue_scatter(b)
            return carry

        lax.fori_loop(0, P, step, 0)
        for b in range(NB):
            wait_scatter(b)
        plsc.subcore_barrier()

        pltpu.sync_copy(acc_sh.at[pl.ds(s * dr, dr)],
                        out_hbm.at[pl.ds(c * half + s * dr, dr)])

    f32 = jnp.float32
    kern = pl.kernel(
        body,
        out_type=jax.ShapeDtypeStruct((np_, d), f32),
        mesh=mesh,
        scratch_types=(
            [pltpu.VMEM_SHARED((nploc, d), f32)]
            + [pltpu.VMEM((2 * C,), jnp.int32)] * NB
            + [pltpu.VMEM((C,), jnp.int32)] * NB
            + [pltpu.VMEM((C, d), f32)] * NB
            + [pltpu.SemaphoreType.DMA] * (3 * NB)
        ),
    )
    return kern(table, packed, zacc)


def _tc_pre(bn, x, Wh):
    n, d = x.shape

    def body(x_ref, Wh_ref, out_ref):
        out_ref[...] = jnp.dot(x_ref[...], Wh_ref[...],
                               preferred_element_type=jnp.float32)

    return pl.pallas_call(
        body,
        grid=(n // bn,),
        in_specs=[
            pl.BlockSpec((bn, d), lambda i: (i, 0)),
            pl.BlockSpec((d, d), lambda i: (0, 0)),
        ],
        out_specs=pl.BlockSpec((bn, d), lambda i: (i, 0)),
        out_shape=jax.ShapeDtypeStruct((n, d), jnp.float32),
    )(x, Wh)


def _tc_final(bn, x, z, w, esel, WeT, be, b0, b1, WcT, bc):
    n, d = x.shape
    o = WcT.shape[1]
    h = WeT.shape[1]

    def body(x_ref, z_ref, w_ref, esel_ref, WeT_ref, be_ref, b0_ref,
             b1_ref, WcT_ref, bc_ref, out_ref):
        he = jax.nn.relu(
            jnp.dot(x_ref[...], WeT_ref[...],
                    preferred_element_type=jnp.float32) + be_ref[...])
        wv = w_ref[...]
        deg = jnp.dot(wv, esel_ref[...], preferred_element_type=jnp.float32)
        dinv = 1.0 / jnp.maximum(deg, 1.0)
        h1 = jax.nn.relu(z_ref[...] * dinv + b0_ref[...])
        h2 = jax.nn.relu(wv[:, 0:h] * (dinv * dinv) + b1_ref[...])
        wc = WcT_ref[...]
        out = (jnp.dot(he, wc[0:h], preferred_element_type=jnp.float32)
               + jnp.dot(h1, wc[h:2 * h], preferred_element_type=jnp.float32)
               + jnp.dot(h2, wc[2 * h:3 * h], preferred_element_type=jnp.float32)
               + bc_ref[...])
        out_ref[...] = out

    full = lambda shape: pl.BlockSpec(shape, lambda i: (0,) * len(shape))
    return pl.pallas_call(
        body,
        grid=(n // bn,),
        in_specs=[
            pl.BlockSpec((bn, d), lambda i: (i, 0)),
            pl.BlockSpec((bn, h), lambda i: (i, 0)),
            pl.BlockSpec((bn, d), lambda i: (i, 0)),
            full((d, 1)),
            full((d, h)), full((1, h)),
            full((1, h)), full((1, h)),
            full((3 * h, o)), full((1, o)),
        ],
        out_specs=pl.BlockSpec((bn, o), lambda i: (i, 0)),
        out_shape=jax.ShapeDtypeStruct((n, o), jnp.float32),
    )(x, z, w, esel, WeT, be, b0, b1, WcT, bc)


def kernel(x, edge_index, W_ego, b_ego, W_hop0, b_hop0, W_hop1, b_hop1,
           W_cls, b_cls):
    n, d = x.shape
    e = edge_index.shape[1]
    h = W_ego.shape[0]
    np_ = -(-(n + 1) // (NC * NS * 8)) * (NC * NS * 8)
    half = np_ // NC

    per_tile = -(-e // (NS * 4 * C)) * 4 * C
    epad = NS * per_tile

    src = edge_index[0].astype(jnp.int32)
    dst = edge_index[1].astype(jnp.int32)
    if epad > e:
        fill = jnp.full((epad - e,), n, dtype=jnp.int32)
        src = jnp.concatenate([src, fill])
        dst = jnp.concatenate([dst, fill])
    packed = jnp.stack([dst.reshape(-1, C), src.reshape(-1, C)],
                       axis=1).reshape(-1)

    nploc = half + C
    zacc = jnp.zeros((nploc, d), jnp.float32)

    Wh = jnp.concatenate([W_hop0.T, W_hop1.T], axis=1)
    y1 = _tc_pre(1000, x, Wh)
    y1p = jnp.concatenate([y1, jnp.zeros((np_ - n, d), jnp.float32)], axis=0)

    z = _sc_hop(y1p, packed, zacc, np_=np_, half=half, epad=epad)
    t2 = jnp.concatenate([z[:, h:], jnp.ones((np_, 1), jnp.float32),
                          jnp.zeros((np_, d - h - 1), jnp.float32)], axis=1)
    w = _sc_hop(t2, packed, zacc, np_=np_, half=half, epad=epad)

    esel = jnp.zeros((d, 1), jnp.float32).at[h, 0].set(1.0)
    return _tc_final(1000, x, z[:n, :h], w[:n], esel,
                     W_ego.T, b_ego[None, :], b_hop0[None, :],
                     b_hop1[None, :], W_cls.T, b_cls[None, :])
